# baseline (speedup 1.0000x reference)
import jax
import jax.numpy as jnp
from jax import lax
from jax.experimental import pallas as pl
from jax.experimental.pallas import tpu as pltpu
from jax.experimental.pallas import tpu_sc as plsc

NUM_STATES = 1000000
NUM_ACTIONS = 64
BATCH = 16384

NC = 2
NS = 16
NW = NC * NS
LANES = 16
BPW = BATCH // NW
NSLABS = (NUM_STATES + 127) // 128
SPT = (NSLABS + NW - 1) // NW
SB = 4
NBLK = (SPT + SB - 1) // SB
NRING = 2
NSTAGE = 8
LCAP = BATCH + LANES

_NEG = -1000000000.0


def _splat(iota16, zeros16, v, j):
    return zeros16 + jnp.sum(jnp.where(iota16 == j, v, 0))


def _elem(iota16, zeros16, ref, p):
    j = lax.rem(p, LANES)
    v = ref[pl.ds(pl.multiple_of(p - j, LANES), LANES)]
    return _splat(iota16, zeros16, v, j)


def _gather_body(obs_hbm, qT_hbm, scr_hbm,
                 obs_v, kl_v, sk_v, hist_v, off_v, off0_v,
                 slab_v, stage_v, gsem, ssem):
    wid = lax.axis_index("s") * NC + lax.axis_index("c")
    lo = wid * SPT
    hi = jnp.minimum(lo + SPT, NSLABS)
    iota16 = lax.iota(jnp.int32, LANES)
    lane0 = iota16 == 0
    zeros16 = iota16 * 0

    pltpu.sync_copy(obs_hbm, obs_v)

    def scan(k, n):
        s16 = obs_v[pl.ds(k * LANES, LANES)]
        c16 = lax.shift_right_logical(s16, 7)
        msk = jnp.logical_and(c16 >= lo, c16 < hi)
        key = (lax.shift_left(c16 - lo, 21)
               | lax.shift_left(lax.bitwise_and(s16, 127), 14)
               | (iota16 + k * LANES))
        plsc.store_compressed(kl_v.at[pl.ds(n, LANES)], key, mask=msk)
        pc = plsc.all_reduce_population_count(msk)
        return n + jnp.sum(jnp.where(lane0, pc, 0))

    n = lax.fori_loop(0, BATCH // LANES, scan, 0, unroll=2)

    for k in range(256 // LANES):
        hist_v[pl.ds(k * LANES, LANES)] = zeros16

    def hist_pass(p, carry):
        kp = _elem(iota16, zeros16, kl_v, p)
        bv = lax.shift_right_logical(kp, 21)
        cnt = plsc.load_gather(hist_v, [bv])
        plsc.store_scatter(hist_v, [bv], cnt + 1, mask=lane0)
        return carry

    lax.fori_loop(0, n, hist_pass, 0)

    run = 0
    for k in range(256 // LANES):
        v = hist_v[pl.ds(k * LANES, LANES)]
        ex = plsc.cumsum(v) - v + run
        off_v[pl.ds(k * LANES, LANES)] = ex
        off0_v[pl.ds(k * LANES, LANES)] = ex
        run = run + jnp.sum(v)

    def scatter_pass(p, carry):
        kp = _elem(iota16, zeros16, kl_v, p)
        bv = lax.shift_right_logical(kp, 21)
        pos16 = plsc.load_gather(off_v, [bv])
        plsc.store_scatter(sk_v, [pos16], kp, mask=lane0)
        plsc.store_scatter(off_v, [bv], pos16 + 1, mask=lane0)
        return carry

    lax.fori_loop(0, n, scatter_pass, 0)

    def blk_bounds(m):
        start = _elem(iota16, zeros16, off0_v, m * SB)
        end = _elem(iota16, zeros16, off0_v, m * SB + SB)
        return (jnp.sum(jnp.where(lane0, start, 0)),
                jnp.sum(jnp.where(lane0, end, 0)))

    def stslab(m):
        return jnp.minimum(lo + m * SB, NSLABS - SB)

    def fire(m):
        st = pl.multiple_of(stslab(m) * 128, 128)
        slot = lax.rem(m, NRING)
        return pltpu.async_copy(
            qT_hbm.at[:, pl.ds(st, SB * 128)], slab_v.at[slot],
            gsem.at[slot])

    def want(m):
        s0, e0 = blk_bounds(m)
        return jnp.logical_and(m < NBLK, e0 > s0)

    for m in range(NRING - 1):
        @pl.when(want(m))
        def _(m=m):
            fire(m)

    def blk_step(m, carry):
        @pl.when(want(m + NRING - 1))
        def _():
            fire(m + NRING - 1)

        start, end = blk_bounds(m)

        @pl.when(end > start)
        def _():
            slot = lax.rem(m, NRING)
            pltpu.make_async_copy(
                qT_hbm.at[:, pl.ds(0, SB * 128)], slab_v.at[slot],
                gsem.at[slot]).wait()
            shift = zeros16 + lax.shift_left(lo - stslab(m), 7)

            def pair_step(i, c2):
                kp = _elem(iota16, zeros16, sk_v, i)
                colw = lax.shift_right_logical(kp, 14) + shift
                b = jnp.sum(jnp.where(lane0, kp & 16383, 0))
                slot2 = lax.rem(i, NSTAGE)

                @pl.when(i >= NSTAGE)
                def _():
                    pltpu.make_async_copy(
                        stage_v.at[slot2],
                        scr_hbm.at[pl.ds(0, NUM_ACTIONS)],
                        ssem.at[slot2]).wait()

                for g in range(NUM_ACTIONS // LANES):
                    q = plsc.load_gather(slab_v.at[slot],
                                         [iota16 + g * LANES, colw])
                    stage_v[slot2, pl.ds(g * LANES, LANES)] = q
                pltpu.async_copy(
                    stage_v.at[slot2],
                    scr_hbm.at[pl.ds(pl.multiple_of(b * NUM_ACTIONS, 8),
                                     NUM_ACTIONS)],
                    ssem.at[slot2])
                return c2

            lax.fori_loop(start, end, pair_step, 0)
        return carry

    lax.fori_loop(0, NBLK, blk_step, 0)

    for j in range(NSTAGE):
        @pl.when(n - NSTAGE + j >= 0)
        def _(j=j):
            slot2 = lax.rem(n - NSTAGE + j, NSTAGE)
            pltpu.make_async_copy(
                stage_v.at[slot2], scr_hbm.at[pl.ds(0, NUM_ACTIONS)],
                ssem.at[slot2]).wait()


def _mask_body(scr_hbm, masksT_hbm, outT_hbm, scr_v, m_v, msem):
    wid = lax.axis_index("s") * NC + lax.axis_index("c")
    base = wid * BPW
    iota16 = lax.iota(jnp.int32, LANES)

    mcopy = pltpu.async_copy(masksT_hbm.at[:, pl.ds(base, BPW)], m_v, msem)
    pltpu.sync_copy(
        scr_hbm.at[pl.ds(base * NUM_ACTIONS, BPW * NUM_ACTIONS)], scr_v)
    mcopy.wait()

    def arow(a, carry):
        for k in range(BPW // LANES):
            sl = pl.ds(k * LANES, LANES)
            idx = lax.shift_left(iota16 + k * LANES, 6) + a
            q = plsc.load_gather(scr_v, [idx])
            m = m_v[a, sl]
            m_v[a, sl] = m * q + (1.0 - m) * _NEG
        return carry

    lax.fori_loop(0, NUM_ACTIONS, arow, 0)

    pltpu.sync_copy(m_v, outT_hbm.at[:, pl.ds(base, BPW)])


def kernel(observations, action_masks, q_table):
    obs = observations.reshape(-1).astype(jnp.int32)
    qT = q_table.T
    masksT = action_masks.T
    mesh = plsc.VectorSubcoreMesh(
        core_axis_name="c", subcore_axis_name="s", num_cores=NC,
        num_subcores=NS)
    params = pltpu.CompilerParams(needs_layout_passes=False)

    gather = pl.kernel(
        _gather_body,
        out_type=jax.ShapeDtypeStruct((BATCH * NUM_ACTIONS,), jnp.float32),
        mesh=mesh,
        scratch_types=[
            pltpu.VMEM((BATCH,), jnp.int32),
            pltpu.VMEM((LCAP,), jnp.int32),
            pltpu.VMEM((LCAP,), jnp.int32),
            pltpu.VMEM((256,), jnp.int32),
            pltpu.VMEM((256,), jnp.int32),
            pltpu.VMEM((256,), jnp.int32),
            pltpu.VMEM((NRING, NUM_ACTIONS, SB * 128), jnp.float32),
            pltpu.VMEM((NSTAGE, NUM_ACTIONS), jnp.float32),
            pltpu.SemaphoreType.DMA((NRING,)),
            pltpu.SemaphoreType.DMA((NSTAGE,)),
        ],
        compiler_params=params,
    )

    masker = pl.kernel(
        _mask_body,
        out_type=jax.ShapeDtypeStruct((NUM_ACTIONS, BATCH), jnp.float32),
        mesh=mesh,
        scratch_types=[
            pltpu.VMEM((BPW * NUM_ACTIONS,), jnp.float32),
            pltpu.VMEM((NUM_ACTIONS, BPW), jnp.float32),
            pltpu.SemaphoreType.DMA,
        ],
        compiler_params=params,
    )

    scratch = gather(obs, qT)
    outT = masker(scratch, masksT)
    return outT.T


# speedup vs baseline: 1.0925x; 1.0925x over previous
import jax
import jax.numpy as jnp
from jax import lax
from jax.experimental import pallas as pl
from jax.experimental.pallas import tpu as pltpu
from jax.experimental.pallas import tpu_sc as plsc

NUM_STATES = 1000000
NUM_ACTIONS = 64
BATCH = 16384

NC = 2
NS = 16
NW = NC * NS
LANES = 16
BPW = BATCH // NW
NSLABS = (NUM_STATES + 127) // 128
SPT = (NSLABS + NW - 1) // NW
SB = 2
NBLK = (SPT + SB - 1) // SB
NRING = 4
NSTAGE = 8
LCAP = BATCH + LANES

_NEG = -1000000000.0


def _splat(iota16, zeros16, v, j):
    return zeros16 + jnp.sum(jnp.where(iota16 == j, v, 0))


def _elem(iota16, zeros16, ref, p):
    j = lax.rem(p, LANES)
    v = ref[pl.ds(pl.multiple_of(p - j, LANES), LANES)]
    return _splat(iota16, zeros16, v, j)


def _gather_body(obs_hbm, qT_hbm, scr_hbm,
                 obs_v, kl_v, sk_v, hist_v, off_v, off0_v,
                 slab_v, stage_v, gsem, ssem):
    wid = lax.axis_index("s") * NC + lax.axis_index("c")
    lo = wid * SPT
    hi = jnp.minimum(lo + SPT, NSLABS)
    iota16 = lax.iota(jnp.int32, LANES)
    lane0 = iota16 == 0
    zeros16 = iota16 * 0

    pltpu.sync_copy(obs_hbm, obs_v)

    def scan(k, n):
        s16 = obs_v[pl.ds(k * LANES, LANES)]
        c16 = lax.shift_right_logical(s16, 7)
        msk = jnp.logical_and(c16 >= lo, c16 < hi)
        key = (lax.shift_left(c16 - lo, 21)
               | lax.shift_left(lax.bitwise_and(s16, 127), 14)
               | (iota16 + k * LANES))
        plsc.store_compressed(kl_v.at[pl.ds(n, LANES)], key, mask=msk)
        pc = plsc.all_reduce_population_count(msk)
        return n + jnp.sum(jnp.where(lane0, pc, 0))

    n = lax.fori_loop(0, BATCH // LANES, scan, 0, unroll=2)

    for k in range(256 // LANES):
        hist_v[pl.ds(k * LANES, LANES)] = zeros16

    def hist_pass(p, carry):
        kp = _elem(iota16, zeros16, kl_v, p)
        bv = lax.shift_right_logical(kp, 21)
        cnt = plsc.load_gather(hist_v, [bv])
        plsc.store_scatter(hist_v, [bv], cnt + 1, mask=lane0)
        return carry

    lax.fori_loop(0, n, hist_pass, 0)

    run = 0
    for k in range(256 // LANES):
        v = hist_v[pl.ds(k * LANES, LANES)]
        ex = plsc.cumsum(v) - v + run
        off_v[pl.ds(k * LANES, LANES)] = ex
        off0_v[pl.ds(k * LANES, LANES)] = ex
        run = run + jnp.sum(v)

    def scatter_pass(p, carry):
        kp = _elem(iota16, zeros16, kl_v, p)
        bv = lax.shift_right_logical(kp, 21)
        pos16 = plsc.load_gather(off_v, [bv])
        plsc.store_scatter(sk_v, [pos16], kp, mask=lane0)
        plsc.store_scatter(off_v, [bv], pos16 + 1, mask=lane0)
        return carry

    lax.fori_loop(0, n, scatter_pass, 0)

    def blk_bounds(m):
        start = _elem(iota16, zeros16, off0_v, m * SB)
        end = _elem(iota16, zeros16, off0_v, m * SB + SB)
        return (jnp.sum(jnp.where(lane0, start, 0)),
                jnp.sum(jnp.where(lane0, end, 0)))

    def stslab(m):
        return jnp.minimum(lo + m * SB, NSLABS - SB)

    def fire(m):
        st = pl.multiple_of(stslab(m) * 128, 128)
        slot = lax.rem(m, NRING)
        return pltpu.async_copy(
            qT_hbm.at[:, pl.ds(st, SB * 128)], slab_v.at[slot],
            gsem.at[slot])

    def want(m):
        s0, e0 = blk_bounds(m)
        return jnp.logical_and(m < NBLK, e0 > s0)

    for m in range(NRING - 1):
        @pl.when(want(m))
        def _(m=m):
            fire(m)

    def blk_step(m, carry):
        @pl.when(want(m + NRING - 1))
        def _():
            fire(m + NRING - 1)

        start, end = blk_bounds(m)

        @pl.when(end > start)
        def _():
            slot = lax.rem(m, NRING)
            pltpu.make_async_copy(
                qT_hbm.at[:, pl.ds(0, SB * 128)], slab_v.at[slot],
                gsem.at[slot]).wait()
            shift = zeros16 + lax.shift_left(lo - stslab(m), 7)

            def pair_step(i, c2):
                kp = _elem(iota16, zeros16, sk_v, i)
                colw = lax.shift_right_logical(kp, 14) + shift
                b = jnp.sum(jnp.where(lane0, kp & 16383, 0))
                slot2 = lax.rem(i, NSTAGE)

                @pl.when(i >= NSTAGE)
                def _():
                    pltpu.make_async_copy(
                        stage_v.at[slot2],
                        scr_hbm.at[pl.ds(0, NUM_ACTIONS)],
                        ssem.at[slot2]).wait()

                for g in range(NUM_ACTIONS // LANES):
                    q = plsc.load_gather(slab_v.at[slot],
                                         [iota16 + g * LANES, colw])
                    stage_v[slot2, pl.ds(g * LANES, LANES)] = q
                pltpu.async_copy(
                    stage_v.at[slot2],
                    scr_hbm.at[pl.ds(pl.multiple_of(b * NUM_ACTIONS, 8),
                                     NUM_ACTIONS)],
                    ssem.at[slot2])
                return c2

            lax.fori_loop(start, end, pair_step, 0)
        return carry

    lax.fori_loop(0, NBLK, blk_step, 0)

    for j in range(NSTAGE):
        @pl.when(n - NSTAGE + j >= 0)
        def _(j=j):
            slot2 = lax.rem(n - NSTAGE + j, NSTAGE)
            pltpu.make_async_copy(
                stage_v.at[slot2], scr_hbm.at[pl.ds(0, NUM_ACTIONS)],
                ssem.at[slot2]).wait()


def _mask_body(scr_hbm, masksT_hbm, outT_hbm, scr_v, m_v, msem):
    wid = lax.axis_index("s") * NC + lax.axis_index("c")
    base = wid * BPW
    iota16 = lax.iota(jnp.int32, LANES)

    mcopy = pltpu.async_copy(masksT_hbm.at[:, pl.ds(base, BPW)], m_v, msem)
    pltpu.sync_copy(
        scr_hbm.at[pl.ds(base * NUM_ACTIONS, BPW * NUM_ACTIONS)], scr_v)
    mcopy.wait()

    def arow(a, carry):
        for k in range(BPW // LANES):
            sl = pl.ds(k * LANES, LANES)
            idx = lax.shift_left(iota16 + k * LANES, 6) + a
            q = plsc.load_gather(scr_v, [idx])
            m = m_v[a, sl]
            m_v[a, sl] = m * q + (1.0 - m) * _NEG
        return carry

    lax.fori_loop(0, NUM_ACTIONS, arow, 0)

    pltpu.sync_copy(m_v, outT_hbm.at[:, pl.ds(base, BPW)])


def kernel(observations, action_masks, q_table):
    obs = observations.reshape(-1).astype(jnp.int32)
    qT = q_table.T
    masksT = action_masks.T
    mesh = plsc.VectorSubcoreMesh(
        core_axis_name="c", subcore_axis_name="s", num_cores=NC,
        num_subcores=NS)
    params = pltpu.CompilerParams(needs_layout_passes=False)

    gather = pl.kernel(
        _gather_body,
        out_type=jax.ShapeDtypeStruct((BATCH * NUM_ACTIONS,), jnp.float32),
        mesh=mesh,
        scratch_types=[
            pltpu.VMEM((BATCH,), jnp.int32),
            pltpu.VMEM((LCAP,), jnp.int32),
            pltpu.VMEM((LCAP,), jnp.int32),
            pltpu.VMEM((256,), jnp.int32),
            pltpu.VMEM((256,), jnp.int32),
            pltpu.VMEM((256,), jnp.int32),
            pltpu.VMEM((NRING, NUM_ACTIONS, SB * 128), jnp.float32),
            pltpu.VMEM((NSTAGE, NUM_ACTIONS), jnp.float32),
            pltpu.SemaphoreType.DMA((NRING,)),
            pltpu.SemaphoreType.DMA((NSTAGE,)),
        ],
        compiler_params=params,
    )

    masker = pl.kernel(
        _mask_body,
        out_type=jax.ShapeDtypeStruct((NUM_ACTIONS, BATCH), jnp.float32),
        mesh=mesh,
        scratch_types=[
            pltpu.VMEM((BPW * NUM_ACTIONS,), jnp.float32),
            pltpu.VMEM((NUM_ACTIONS, BPW), jnp.float32),
            pltpu.SemaphoreType.DMA,
        ],
        compiler_params=params,
    )

    scratch = gather(obs, qT)
    outT = masker(scratch, masksT)
    return outT.T
